# baseline (speedup 1.0000x reference)
import jax
import jax.numpy as jnp
from jax import lax
from jax.experimental import pallas as pl
from jax.experimental.pallas import tpu as pltpu
from jax.experimental.pallas import tpu_sc as plsc

C = 128
DEG = 32
NW = 32
CPT = C // NW
G = 1024
NB = 10
N_PAD = NB * G
L = 16


def _matmul_t_kernel(w_ref, x_ref, o_ref, mx_ref):
  m = lax.dot_general(w_ref[...], x_ref[...], (((0,), (1,)), ((), ())),
                      preferred_element_type=jnp.float32)
  o_ref[...] = m
  mx_ref[...] = jnp.full((1, 1, C), jnp.max(jnp.abs(m)), jnp.float32)


def _gru_kernel(st_ref, x_ref, wih_ref, whh_ref, bih_ref, bhh_ref, inv_ref,
                o_ref):
  h = x_ref[...]
  s = st_ref[...].astype(jnp.float32) * inv_ref[...]
  gi = lax.dot_general(s, wih_ref[...], (((0,), (1,)), ((), ())),
                       preferred_element_type=jnp.float32) + bih_ref[...]
  gh = lax.dot_general(h, whh_ref[...], (((1,), (1,)), ((), ())),
                       preferred_element_type=jnp.float32) + bhh_ref[...]
  r = jax.nn.sigmoid(gi[:, :C] + gh[:, :C])
  z = jax.nn.sigmoid(gi[:, C:2 * C] + gh[:, C:2 * C])
  n = jnp.tanh(gi[:, 2 * C:] + r * gh[:, 2 * C:])
  o_ref[...] = (1.0 - z) * n + z * h


EBLK = G * DEG


PPT = 2


def _gather_sum_body(mt_hbm, e_hbm, out_hbm, tab_v, ebuf_v, obuf_v,
                     sem_t, sem_e, sem_o):
  cix = lax.axis_index("c")
  six = lax.axis_index("s")
  wid = six * 2 + cix
  p0 = wid * PPT
  orow = [p0, p0 + C // 2, p0 + 1, p0 + 1 + C // 2]

  for r in range(PPT):
    pltpu.async_copy(mt_hbm.at[p0 + r], tab_v.at[pl.ds(r * N_PAD, N_PAD)],
                     sem_t)
  pltpu.async_copy(e_hbm.at[0], ebuf_v.at[pl.ds(0, EBLK)], sem_e)
  for r in range(PPT):
    pltpu.make_async_copy(mt_hbm.at[p0 + r],
                          tab_v.at[pl.ds(r * N_PAD, N_PAD)], sem_t).wait()

  offr = [jnp.full((L,), r * N_PAD, jnp.int32) for r in range(PPT)]
  sh16 = jnp.full((L,), 16, jnp.int32)

  def do_block(b, k):
    pltpu.make_async_copy(e_hbm.at[b], ebuf_v.at[pl.ds(k * EBLK, EBLK)],
                          sem_e).wait()

    @pl.when(b + 1 < NB)
    def _():
      pltpu.async_copy(e_hbm.at[b + 1],
                       ebuf_v.at[pl.ds((1 - k) * EBLK, EBLK)], sem_e)

    def g_body(g, carry):
      ebase = k * EBLK + g * (DEG * L)
      obase = k * (CPT * G) + g * L
      acc = [jnp.zeros((L,), jnp.int32) for _ in range(2 * PPT)]
      for d in range(DEG):
        idx = ebuf_v[pl.ds(ebase + d * L, L)]
        for r in range(PPT):
          gv = plsc.load_gather(tab_v, [idx + offr[r]])
          acc[2 * r] = acc[2 * r] + lax.shift_right_arithmetic(gv, sh16)
          acc[2 * r + 1] = acc[2 * r + 1] + lax.shift_right_arithmetic(
              lax.shift_left(gv, sh16), sh16)
      for j in range(2 * PPT):
        obuf_v[pl.ds(obase + j * G, L)] = acc[j]
      return carry

    lax.fori_loop(0, G // L, g_body, 0)
    for j in range(2 * PPT):
      pltpu.async_copy(obuf_v.at[pl.ds(k * (CPT * G) + j * G, G)],
                       out_hbm.at[orow[j], pl.ds(b * G, G)], sem_o)

  def pair(bb, carry):
    for k in range(2):
      b = bb * 2 + k

      @pl.when(bb > 0)
      def _():
        for j in range(2 * PPT):
          pltpu.make_async_copy(
              obuf_v.at[pl.ds(k * (CPT * G) + j * G, G)],
              out_hbm.at[orow[j], pl.ds(b * G, G)], sem_o).wait()

      do_block(b, k)
    return carry

  lax.fori_loop(0, NB // 2, pair, 0)
  for k in range(2):
    b = NB - 2 + k
    for j in range(2 * PPT):
      pltpu.make_async_copy(
          obuf_v.at[pl.ds(k * (CPT * G) + j * G, G)],
          out_hbm.at[orow[j], pl.ds(b * G, G)], sem_o).wait()


def _make_gather_sum():
  mesh = plsc.VectorSubcoreMesh(core_axis_name="c", subcore_axis_name="s")
  return pl.kernel(
      _gather_sum_body,
      out_type=jax.ShapeDtypeStruct((C, N_PAD), jnp.int32),
      mesh=mesh,
      scratch_types=[
          pltpu.VMEM((PPT * N_PAD,), jnp.int32),
          pltpu.VMEM((2 * EBLK,), jnp.int32),
          pltpu.VMEM((2 * CPT * G,), jnp.int32),
          pltpu.SemaphoreType.DMA,
          pltpu.SemaphoreType.DMA,
          pltpu.SemaphoreType.DMA,
      ],
      compiler_params=pltpu.CompilerParams(needs_layout_passes=False),
  )


@jax.jit
def kernel(x, edge_index, weight, W_ih, W_hh, b_ih, b_hh):
  n = x.shape[0]
  x_pad = jnp.zeros((N_PAD, C), jnp.float32).at[:n].set(x)
  e = edge_index.astype(jnp.int32)
  e_pad = jnp.full((N_PAD, DEG), n, jnp.int32).at[:n].set(e)
  e_blk = e_pad.reshape(NB, G // L, L, DEG).transpose(0, 1, 3, 2).reshape(
      NB, EBLK)

  m_t, mx = pl.pallas_call(
      _matmul_t_kernel,
      grid=(NB,),
      in_specs=[
          pl.BlockSpec((C, C), lambda i: (0, 0)),
          pl.BlockSpec((G, C), lambda i: (i, 0)),
      ],
      out_specs=[
          pl.BlockSpec((C, G), lambda i: (0, i)),
          pl.BlockSpec((1, 1, C), lambda i: (i, 0, 0)),
      ],
      out_shape=[
          jax.ShapeDtypeStruct((C, N_PAD), jnp.float32),
          jax.ShapeDtypeStruct((NB, 1, C), jnp.float32),
      ],
  )(weight[0], x_pad)

  amax = jnp.max(mx)
  scale = jnp.where(amax > 0, 32704.0 / amax, 1.0)
  inv = jnp.where(amax > 0, amax / 32704.0, 1.0).reshape(1, 1)
  q = jnp.round(m_t * scale).astype(jnp.int32)
  pk = lax.shift_left(q[:C // 2], 16) | (q[C // 2:] & 0xFFFF)

  s_t = _make_gather_sum()(pk, e_blk)

  out = pl.pallas_call(
      _gru_kernel,
      grid=(NB,),
      in_specs=[
          pl.BlockSpec((C, G), lambda i: (0, i)),
          pl.BlockSpec((G, C), lambda i: (i, 0)),
          pl.BlockSpec((3 * C, C), lambda i: (0, 0)),
          pl.BlockSpec((3 * C, C), lambda i: (0, 0)),
          pl.BlockSpec((1, 3 * C), lambda i: (0, 0)),
          pl.BlockSpec((1, 3 * C), lambda i: (0, 0)),
          pl.BlockSpec((1, 1), lambda i: (0, 0)),
      ],
      out_specs=pl.BlockSpec((G, C), lambda i: (i, 0)),
      out_shape=jax.ShapeDtypeStruct((N_PAD, C), jnp.float32),
  )(s_t, x_pad, W_ih, W_hh, b_ih.reshape(1, 3 * C), b_hh.reshape(1, 3 * C),
    inv)

  return out[:n]
